# baseline (speedup 1.0000x reference)
import jax
import jax.numpy as jnp
from jax.experimental import pallas as pl
from jax.experimental.pallas import tpu as pltpu

_BM = 200


def _gcn_body(t_ref, x_ref, f_ref, o_ref):
    tx = jnp.dot(t_ref[...], x_ref[...], preferred_element_type=jnp.float32)
    o_ref[...] = jnp.dot(tx, f_ref[...], preferred_element_type=jnp.float32)


def kernel(transform, x, filters):
    n, d = x.shape
    nf = filters.shape[1]
    return pl.pallas_call(
        _gcn_body,
        grid=(n // _BM,),
        in_specs=[
            pl.BlockSpec((_BM, n), lambda i: (i, 0)),
            pl.BlockSpec((n, d), lambda i: (0, 0)),
            pl.BlockSpec((d, nf), lambda i: (0, 0)),
        ],
        out_specs=pl.BlockSpec((_BM, nf), lambda i: (i, 0)),
        out_shape=jax.ShapeDtypeStruct((n, nf), jnp.float32),
        compiler_params=pltpu.CompilerParams(
            dimension_semantics=("parallel",),
        ),
    )(transform, x, filters)


# speedup vs baseline: 1.0172x; 1.0172x over previous
import jax
import jax.numpy as jnp
from jax.experimental import pallas as pl
from jax.experimental.pallas import tpu as pltpu

_BM = 400


def _gcn_body(t_ref, x_ref, f_ref, o_ref):
    tx = jnp.dot(t_ref[...], x_ref[...], preferred_element_type=jnp.float32)
    o_ref[...] = jnp.dot(tx, f_ref[...], preferred_element_type=jnp.float32)


def kernel(transform, x, filters):
    n, d = x.shape
    nf = filters.shape[1]
    return pl.pallas_call(
        _gcn_body,
        grid=(n // _BM,),
        in_specs=[
            pl.BlockSpec((_BM, n), lambda i: (i, 0)),
            pl.BlockSpec((n, d), lambda i: (0, 0)),
            pl.BlockSpec((d, nf), lambda i: (0, 0)),
        ],
        out_specs=pl.BlockSpec((_BM, nf), lambda i: (i, 0)),
        out_shape=jax.ShapeDtypeStruct((n, nf), jnp.float32),
        compiler_params=pltpu.CompilerParams(
            dimension_semantics=("parallel",),
        ),
    )(transform, x, filters)
